# Initial kernel scaffold; baseline (speedup 1.0000x reference)
import jax
import jax.numpy as jnp
from jax.experimental import pallas as pl


def kernel(query, support, symbol_emb):
    raise NotImplementedError("write your pallas kernel here")


# speedup vs baseline: 7.5988x; 7.5988x over previous
import functools

import jax
import jax.numpy as jnp
from jax import lax
from jax.experimental import pallas as pl
from jax.experimental.pallas import tpu as pltpu
from jax.experimental.pallas import tpu_sc as plsc

NUM_SYMBOLS = 100000
D = 128
DC = D // 16
B = 16384
S = 64
NC, NS = 2, 16
NW = NC * NS
QPW = B // NW
CH = 128
NCHUNK = QPW // CH
EPS2 = 1e-16


_GATHER_DNUMS = lax.GatherDimensionNumbers(
    offset_dims=(), collapsed_slice_dims=(0,), start_index_map=(0,))


def _lane_shuffle(v, idx):
    return lax.gather(v, idx[:, None], dimension_numbers=_GATHER_DNUMS,
                      slice_sizes=(1,),
                      mode=lax.GatherScatterMode.PROMISE_IN_BOUNDS)


def _hsum(v):
    lane = lax.iota(jnp.int32, 16)
    for off in (8, 4, 2, 1):
        v = v + _lane_shuffle(v, lane ^ off)
    return v


def _rsqrt(x):
    i = lax.bitcast_convert_type(x, jnp.int32)
    i = jnp.int32(0x5F3759DF) - (i >> 1)
    y = lax.bitcast_convert_type(i, jnp.float32)
    for _ in range(3):
        y = y * (1.5 - 0.5 * x * y * y)
    return y


def _sc_body(table, q0, q1, sup, out, sup_idx_v, sup_rows_v,
             idx0_v, idx1_v, rows0_v, rows1_v, out_v, sem):
    wid = lax.axis_index("s") * NC + lax.axis_index("c")
    base = wid * QPW

    pltpu.sync_copy(sup, sup_idx_v)
    pltpu.async_copy(table.at[sup_idx_v], sup_rows_v, sem).wait()

    zeros = jnp.zeros((16,), jnp.float32)

    def sup_body(j, accs):
        new = []
        for k in range(DC):
            new.append(accs[k] + sup_rows_v[j, pl.ds(k * 16, 16)])
        for k in range(DC):
            new.append(accs[DC + k] + sup_rows_v[S + j, pl.ds(k * 16, 16)])
        return tuple(new)

    accs = lax.fori_loop(0, S, sup_body, (zeros,) * (2 * DC))
    m = [a * (1.0 / S) for a in accs]

    msq = zeros
    for k in range(2 * DC):
        msq = msq + m[k] * m[k]
    rs_s = _rsqrt(jnp.maximum(_hsum(msq), EPS2))

    lane = lax.iota(jnp.int32, 16)

    def chunk_body(c, _):
        start = base + c * CH
        pltpu.sync_copy(q0.at[pl.ds(start, CH)], idx0_v)
        pltpu.sync_copy(q1.at[pl.ds(start, CH)], idx1_v)
        cp0 = pltpu.async_copy(table.at[idx0_v], rows0_v, sem)
        cp1 = pltpu.async_copy(table.at[idx1_v], rows1_v, sem)
        cp0.wait()
        cp1.wait()

        def blk_body(j16, _):
            numvec = zeros
            sqvec = zeros
            for l in range(16):
                j = j16 * 16 + l
                num = zeros
                sq = zeros
                for k in range(DC):
                    r0 = rows0_v[j, pl.ds(k * 16, 16)]
                    r1 = rows1_v[j, pl.ds(k * 16, 16)]
                    num = num + r0 * m[k] + r1 * m[DC + k]
                    sq = sq + r0 * r0 + r1 * r1
                sel = lane == l
                numvec = jnp.where(sel, _hsum(num), numvec)
                sqvec = jnp.where(sel, _hsum(sq), sqvec)
            res = numvec * _rsqrt(jnp.maximum(sqvec, EPS2)) * rs_s
            out_v[pl.ds(j16 * 16, 16)] = res
            return 0

        lax.fori_loop(0, CH // 16, blk_body, 0)
        pltpu.sync_copy(out_v, out.at[pl.ds(start, CH)])
        return 0

    lax.fori_loop(0, NCHUNK, chunk_body, 0)


@functools.partial(jax.jit, donate_argnums=())
def _run(table, q0, q1, sup):
    mesh = plsc.VectorSubcoreMesh(core_axis_name="c", subcore_axis_name="s",
                                  num_cores=NC, num_subcores=NS)
    return pl.kernel(
        _sc_body,
        out_type=jax.ShapeDtypeStruct((B,), jnp.float32),
        mesh=mesh,
        scratch_types=[
            pltpu.VMEM((2 * S,), jnp.int32),
            pltpu.VMEM((2 * S, D), jnp.float32),
            pltpu.VMEM((CH,), jnp.int32),
            pltpu.VMEM((CH,), jnp.int32),
            pltpu.VMEM((CH, D), jnp.float32),
            pltpu.VMEM((CH, D), jnp.float32),
            pltpu.VMEM((CH,), jnp.float32),
            pltpu.SemaphoreType.DMA,
        ],
    )(table, q0, q1, sup)


def kernel(query, support, symbol_emb):
    q = query.astype(jnp.int32)
    sup = support.astype(jnp.int32).T.reshape(2 * S)
    return _run(symbol_emb, q[:, 0], q[:, 1], sup)


# speedup vs baseline: 8.3483x; 1.0986x over previous
import functools

import jax
import jax.numpy as jnp
from jax import lax
from jax.experimental import pallas as pl
from jax.experimental.pallas import tpu as pltpu
from jax.experimental.pallas import tpu_sc as plsc

NUM_SYMBOLS = 100000
D = 128
DC = D // 16
B = 16384
S = 64
NC, NS = 2, 16
NW = NC * NS
QPW = B // NW
CH = 128
NCHUNK = QPW // CH
EPS2 = 1e-16


_GATHER_DNUMS = lax.GatherDimensionNumbers(
    offset_dims=(), collapsed_slice_dims=(0,), start_index_map=(0,))


def _lane_shuffle(v, idx):
    return lax.gather(v, idx[:, None], dimension_numbers=_GATHER_DNUMS,
                      slice_sizes=(1,),
                      mode=lax.GatherScatterMode.PROMISE_IN_BOUNDS)


def _hsum(v):
    lane = lax.iota(jnp.int32, 16)
    for off in (8, 4, 2, 1):
        v = v + _lane_shuffle(v, lane ^ off)
    return v


def _rsqrt(x):
    i = lax.bitcast_convert_type(x, jnp.int32)
    i = jnp.int32(0x5F3759DF) - (i >> 1)
    y = lax.bitcast_convert_type(i, jnp.float32)
    for _ in range(3):
        y = y * (1.5 - 0.5 * x * y * y)
    return y


def _sc_body(table, q0, q1, sup, out, sup_idx_v, sup_rows_v,
             idx00_v, idx01_v, idx10_v, idx11_v,
             rows00_v, rows01_v, rows10_v, rows11_v,
             out_v, ssem, sem0, sem1):
    wid = lax.axis_index("s") * NC + lax.axis_index("c")
    base = wid * QPW

    bufs = [(idx00_v, idx01_v, rows00_v, rows01_v, sem0),
            (idx10_v, idx11_v, rows10_v, rows11_v, sem1)]

    def fire(c, buf):
        i0, i1, r0, r1, sem = buf
        start = base + c * CH
        pltpu.sync_copy(q0.at[pl.ds(start, CH)], i0)
        pltpu.sync_copy(q1.at[pl.ds(start, CH)], i1)
        cp0 = pltpu.async_copy(table.at[i0], r0, sem)
        cp1 = pltpu.async_copy(table.at[i1], r1, sem)
        return cp0, cp1

    pltpu.sync_copy(sup, sup_idx_v)
    sup_cp = pltpu.async_copy(table.at[sup_idx_v], sup_rows_v, ssem)
    pending = fire(0, bufs[0])
    sup_cp.wait()

    zeros = jnp.zeros((16,), jnp.float32)

    def sup_body(j, accs):
        new = []
        for k in range(DC):
            new.append(accs[k] + sup_rows_v[j, pl.ds(k * 16, 16)])
        for k in range(DC):
            new.append(accs[DC + k] + sup_rows_v[S + j, pl.ds(k * 16, 16)])
        return tuple(new)

    accs = lax.fori_loop(0, S, sup_body, (zeros,) * (2 * DC))
    m = [a * (1.0 / S) for a in accs]

    msq = zeros
    for k in range(2 * DC):
        msq = msq + m[k] * m[k]
    rs_s = _rsqrt(jnp.maximum(_hsum(msq), EPS2))

    lane = lax.iota(jnp.int32, 16)

    def compute(c, buf):
        _, _, rows0_v, rows1_v, _ = buf

        def blk_body(j16, _):
            numvec = zeros
            sqvec = zeros
            for l in range(16):
                j = j16 * 16 + l
                num = zeros
                sq = zeros
                for k in range(DC):
                    r0 = rows0_v[j, pl.ds(k * 16, 16)]
                    r1 = rows1_v[j, pl.ds(k * 16, 16)]
                    num = num + r0 * m[k] + r1 * m[DC + k]
                    sq = sq + r0 * r0 + r1 * r1
                sel = lane == l
                numvec = jnp.where(sel, _hsum(num), numvec)
                sqvec = jnp.where(sel, _hsum(sq), sqvec)
            res = numvec * _rsqrt(jnp.maximum(sqvec, EPS2)) * rs_s
            out_v[pl.ds(j16 * 16, 16)] = res
            return 0

        lax.fori_loop(0, CH // 16, blk_body, 0)
        pltpu.sync_copy(out_v, out.at[pl.ds(base + c * CH, CH)])

    for c in range(NCHUNK):
        nxt = fire(c + 1, bufs[(c + 1) % 2]) if c + 1 < NCHUNK else None
        for cp in pending:
            cp.wait()
        compute(c, bufs[c % 2])
        pending = nxt


@functools.partial(jax.jit, donate_argnums=())
def _run(table, q0, q1, sup):
    mesh = plsc.VectorSubcoreMesh(core_axis_name="c", subcore_axis_name="s",
                                  num_cores=NC, num_subcores=NS)
    return pl.kernel(
        _sc_body,
        out_type=jax.ShapeDtypeStruct((B,), jnp.float32),
        mesh=mesh,
        scratch_types=[
            pltpu.VMEM((2 * S,), jnp.int32),
            pltpu.VMEM((2 * S, D), jnp.float32),
            pltpu.VMEM((CH,), jnp.int32),
            pltpu.VMEM((CH,), jnp.int32),
            pltpu.VMEM((CH,), jnp.int32),
            pltpu.VMEM((CH,), jnp.int32),
            pltpu.VMEM((CH, D), jnp.float32),
            pltpu.VMEM((CH, D), jnp.float32),
            pltpu.VMEM((CH, D), jnp.float32),
            pltpu.VMEM((CH, D), jnp.float32),
            pltpu.VMEM((CH,), jnp.float32),
            pltpu.SemaphoreType.DMA,
            pltpu.SemaphoreType.DMA,
            pltpu.SemaphoreType.DMA,
        ],
    )(table, q0, q1, sup)


def kernel(query, support, symbol_emb):
    q = query.astype(jnp.int32)
    sup = support.astype(jnp.int32).T.reshape(2 * S)
    return _run(symbol_emb, q[:, 0], q[:, 1], sup)
